# baseline (speedup 1.0000x reference)
import numpy as np
import jax
import jax.numpy as jnp
from jax.experimental import pallas as pl

N_BATCH = 16
N_CHAN = 512
N_COL = 4096
SL = 32
LN = 128

CB = 64
NCB = N_CHAN // CB


def _body(in_ref, out_ref):
    for j in range(CB):
        out_ref[0, j] = in_ref[0, CB - 1 - j]


def kernel(x, cond):
    del cond
    x4 = x.reshape(N_BATCH, N_CHAN, SL, LN)
    z4 = pl.pallas_call(
        _body,
        grid=(N_BATCH, NCB),
        in_specs=[
            pl.BlockSpec((1, CB, SL, LN),
                         lambda b, c: (b, NCB - 1 - c, 0, 0)),
        ],
        out_specs=pl.BlockSpec((1, CB, SL, LN), lambda b, c: (b, c, 0, 0)),
        out_shape=jax.ShapeDtypeStruct((N_BATCH, N_CHAN, SL, LN),
                                       jnp.float32),
    )(x4)
    log_det_J = jnp.zeros((1,), dtype=jnp.float32)
    return (z4.reshape(N_BATCH, N_CHAN, N_COL), log_det_J)


# speedup vs baseline: 2.7295x; 2.7295x over previous
import numpy as np
import jax
import jax.numpy as jnp
from jax import lax
from jax.experimental import pallas as pl
from jax.experimental.pallas import tpu as pltpu

N_BATCH = 16
N_CHAN = 512
N_COL = 4096

CB = 64
NCB = N_CHAN // CB


def _flip8(g, bit1, bit2):
    g = pltpu.roll(g, 4, 0)
    g = jnp.where(bit2, pltpu.roll(g, 6, 0), pltpu.roll(g, 2, 0))
    g = jnp.where(bit1, pltpu.roll(g, 7, 0), pltpu.roll(g, 1, 0))
    return g


def _body(in_ref, out_ref):
    i = lax.broadcasted_iota(jnp.int32, (8, N_COL), 0)
    bit1 = (i & 1) != 0
    bit2 = (i & 2) != 0
    for j in range(CB // 8):
        g = in_ref[0, (CB // 8 - 1 - j) * 8:(CB // 8 - j) * 8, :]
        out_ref[0, j * 8:(j + 1) * 8, :] = _flip8(g, bit1, bit2)


def kernel(x, cond):
    del cond
    z = pl.pallas_call(
        _body,
        grid=(N_BATCH, NCB),
        in_specs=[
            pl.BlockSpec((1, CB, N_COL), lambda b, c: (b, NCB - 1 - c, 0)),
        ],
        out_specs=pl.BlockSpec((1, CB, N_COL), lambda b, c: (b, c, 0)),
        out_shape=jax.ShapeDtypeStruct((N_BATCH, N_CHAN, N_COL),
                                       jnp.float32),
    )(x)
    log_det_J = jnp.zeros((1,), dtype=jnp.float32)
    return (z, log_det_J)
